# baseline (speedup 1.0000x reference)
import functools

import jax
import jax.numpy as jnp
from jax import lax
from jax.experimental import pallas as pl
from jax.experimental.pallas import tpu as pltpu
from jax.experimental.pallas import tpu_sc as plsc


_CH = 128


def _make_sc_agg(n_nodes: int, n_edges_padded: int):
    CH = _CH
    NTILES = 16
    NGTOT = n_edges_padded // CH
    assert NGTOT * CH == n_edges_padded
    NPT = ((n_nodes // NTILES + 7) // 8) * 8
    ACC = NPT + 8
    LAST = n_nodes - NPT * (NTILES - 1)
    assert 0 < LAST <= NPT and LAST % 8 == 0

    mesh = plsc.VectorSubcoreMesh(core_axis_name="c", subcore_axis_name="s")

    @functools.partial(
        pl.kernel,
        mesh=mesh,
        out_type=jax.ShapeDtypeStruct((4, n_nodes, 128), jnp.float32),
        scratch_types=[
            pltpu.VMEM((ACC, 128), jnp.float32),
            pltpu.VMEM((CH,), jnp.int32),
            pltpu.VMEM((CH,), jnp.int32),
            pltpu.VMEM((CH, 128), jnp.float32),
            pltpu.VMEM((CH, 128), jnp.float32),
            pltpu.VMEM((16,), jnp.int32),
            pltpu.VMEM((CH,), jnp.int32),
            pltpu.VMEM((CH,), jnp.int32),
            pltpu.SemaphoreType.DMA,
            pltpu.SemaphoreType.DMA,
            pltpu.SemaphoreType.DMA,
            pltpu.SemaphoreType.DMA,
            pltpu.SemaphoreType.DMA,
            pltpu.SemaphoreType.DMA,
        ],
    )
    def sc_agg(h_hbm, src_hbm, dst_hbm, bnd_hbm, out_hbm,
               acc, sidx0, sidx1, rows0, rows1,
               bndv, sm_d0, sm_d1,
               isem0, isem1, dsem0, dsem1, gsem0, gsem1):
        cid = lax.axis_index("c")
        sid = lax.axis_index("s")
        rowb = pl.multiple_of(sid * NPT, 8)

        pltpu.sync_copy(bnd_hbm.at[sid], bndv)
        bvec = bndv[...]
        e0 = bvec[0]
        e1 = bvec[1]
        g_lo = e0 // CH
        cnt = (e1 + CH - 1) // CH - g_lo
        gmax = NGTOT - 1

        def gcl(g):
            return jnp.minimum(g, gmax)

        def _sidx_desc(g, sb, sem):
            off = pl.multiple_of(gcl(g) * CH, 8)
            return pltpu.make_async_copy(src_hbm.at[pl.ds(off, CH)], sb, sem)

        def _smd_desc(g, db, sem):
            off = pl.multiple_of(gcl(g) * CH, 8)
            return pltpu.make_async_copy(dst_hbm.at[pl.ds(off, CH)], db, sem)

        def sidx_start(g, sb, sem):
            _sidx_desc(g, sb, sem).start()

        def sidx_wait(g, sb, sem):
            _sidx_desc(g, sb, sem).wait()

        def smd_start(g, db, sem):
            _smd_desc(g, db, sem).start()

        def smd_wait(g, db, sem):
            _smd_desc(g, db, sem).wait()

        def do_chunk(f):

            def g_start(sb, rb, sem):
                pltpu.async_copy(h_hbm.at[f].at[sb], rb, sem)

            def g_wait(sb, rb, sem):
                pltpu.make_async_copy(h_hbm.at[f].at[sb], rb, sem).wait()

            def zr(i, c):
                for k in range(8):
                    acc[i, pl.ds(k * 16, 16)] = jnp.zeros((16,), jnp.float32)
                return c
            lax.fori_loop(0, ACC, zr, 0)

            def proc(g, rowsX, dbX, carry):
                gb = g * CH

                def grp(t, c):
                    i0 = pl.multiple_of(t * 16, 16)
                    dvec = dbX[pl.ds(i0, 16)]
                    for j in range(16):
                        prev = c[0]
                        d = dvec[j]
                        i = i0 + j
                        eg = gb + i
                        valid = jnp.logical_and(eg >= e0, eg < e1)
                        dl = jnp.where(valid, d - rowb, NPT)
                        m = jnp.where(d == prev, 1.0, 0.0)
                        out = [jnp.where(valid, d, jnp.int32(-1))]
                        for k in range(8):
                            r = rowsX[i, pl.ds(k * 16, 16)]
                            a = c[k + 1] * m + r
                            acc[dl, pl.ds(k * 16, 16)] = a
                            out.append(a)
                        c = tuple(out)
                    return c
                return lax.fori_loop(0, CH // 16, grp, carry)

            sidx_start(g_lo, sidx0, isem0)
            smd_start(g_lo, sm_d0, dsem0)
            sidx_wait(g_lo, sidx0, isem0)
            g_start(sidx0, rows0, gsem0)
            sidx_start(g_lo + 1, sidx1, isem1)
            smd_start(g_lo + 1, sm_d1, dsem1)
            smd_wait(g_lo, sm_d0, dsem0)

            init = (jnp.int32(-1),) + tuple(
                jnp.zeros((16,), jnp.float32) for _ in range(8))

            def pair(i, carry):
                g = g_lo + 2 * i
                sidx_wait(g + 1, sidx1, isem1)
                g_start(sidx1, rows1, gsem1)
                g_wait(sidx0, rows0, gsem0)
                sidx_start(g + 2, sidx0, isem0)
                carry = proc(g, rows0, sm_d0, carry)
                smd_start(g + 2, sm_d0, dsem0)
                sidx_wait(g + 2, sidx0, isem0)
                g_start(sidx0, rows0, gsem0)
                g_wait(sidx1, rows1, gsem1)
                sidx_start(g + 3, sidx1, isem1)
                smd_wait(g + 1, sm_d1, dsem1)
                carry = proc(g + 1, rows1, sm_d1, carry)
                smd_start(g + 3, sm_d1, dsem1)
                smd_wait(g + 2, sm_d0, dsem0)
                return carry
            carry = lax.fori_loop(0, cnt // 2, pair, init)

            @pl.when(cnt % 2 == 1)
            def _():
                g = g_lo + cnt - 1
                g_wait(sidx0, rows0, gsem0)
                proc(g, rows0, sm_d0, carry)
                sidx_wait(g + 1, sidx1, isem1)
                smd_wait(g + 1, sm_d1, dsem1)

            @pl.when(cnt % 2 == 0)
            def _():
                g_wait(sidx0, rows0, gsem0)
                sidx_wait(0, sidx1, isem1)
                smd_wait(0, sm_d1, dsem1)

            @pl.when(sid < NTILES - 1)
            def _():
                pltpu.sync_copy(acc.at[pl.ds(0, NPT)],
                                out_hbm.at[f].at[pl.ds(rowb, NPT)])

            @pl.when(sid == NTILES - 1)
            def _():
                pltpu.sync_copy(acc.at[pl.ds(0, LAST)],
                                out_hbm.at[f].at[pl.ds(rowb, LAST)])

        @pl.when(cid == 0)
        def _():
            do_chunk(0)
            do_chunk(1)

        @pl.when(cid == 1)
        def _():
            do_chunk(2)
            do_chunk(3)

    return sc_agg


def _init_body(x_ref, w_ref, b_ref, out_ref):
    h = jnp.dot(x_ref[...], w_ref[...], preferred_element_type=jnp.float32)
    h = h + b_ref[...]
    for f in range(4):
        out_ref[f] = h[:, 128 * f:128 * (f + 1)]


def _stats_body(ep_ref, h_ref, a_ref, stat_ref):
    u = ep_ref[...] * h_ref[...] + a_ref[...]
    s = jnp.sum(u, axis=1)
    q = jnp.sum(u * u, axis=1)

    @pl.when(pl.program_id(0) == 0)
    def _():
        stat_ref[...] = jnp.zeros_like(stat_ref)

    stat_ref[:, 0, :] += s
    stat_ref[:, 1, :] += q


def _norm_body(n_nodes, last, ep_ref, stat_ref, g_ref, bt_ref, h_ref, a_ref,
               out_ref, gacc_ref):
    u = ep_ref[...] * h_ref[...] + a_ref[...]
    mean = stat_ref[:, 0:1, :] / n_nodes
    var = stat_ref[:, 1:2, :] / n_nodes - mean * mean
    scale = g_ref[...] * lax.rsqrt(var + 1e-5)
    r = jnp.maximum((u - mean) * scale + bt_ref[...], 0.0)
    out_ref[...] = r

    if last:
        @pl.when(pl.program_id(0) == 0)
        def _():
            gacc_ref[...] = jnp.zeros_like(gacc_ref)
        gacc_ref[:, 0, :] += jnp.sum(r, axis=1)


def _head_body(gp_ref, w0_ref, b0_ref, w1_ref, b1_ref, wp_ref, bp_ref,
               out_ref):
    g = jnp.sum(gp_ref[...], axis=1)
    acc = jnp.zeros((1, w0_ref.shape[2]), jnp.float32)
    for f in range(4):
        acc = acc + jnp.dot(g[f:f + 1, :], w0_ref[f],
                            preferred_element_type=jnp.float32)
    h1 = jnp.maximum(acc + b0_ref[...], 0.0)
    h2 = jnp.maximum(jnp.dot(h1, w1_ref[...],
                             preferred_element_type=jnp.float32) + b1_ref[...],
                     0.0)
    out_ref[...] = jnp.dot(h2, wp_ref[...],
                           preferred_element_type=jnp.float32) + bp_ref[...]


def kernel(x, edge_index, W_init, b_init, eps, bn_gamma, bn_beta,
           W_fc, b_fc, W_pred, b_pred):
    n, d_in = x.shape
    h_dim = W_init.shape[1]
    n_layers = eps.shape[0]
    n_edges = edge_index.shape[1]
    n_out = W_pred.shape[1]
    assert h_dim == 512

    CH = _CH
    kb = max(n - 1, 1).bit_length()
    assert n << kb < 2 ** 31
    packed = jnp.sort((edge_index[1] << kb) | edge_index[0])
    src_s = packed & ((1 << kb) - 1)
    dst_s = packed >> kb
    e_pad = -(-n_edges // CH) * CH
    pad = e_pad - n_edges
    src1d = jnp.concatenate([src_s, jnp.zeros((pad,), jnp.int32)])
    dst1d = jnp.concatenate([dst_s, jnp.full((pad,), n, jnp.int32)])
    npt = ((n // 16 + 7) // 8) * 8
    tick = jnp.arange(16, dtype=jnp.int32) * npt
    starts = jnp.searchsorted(dst1d, tick).astype(jnp.int32)
    ends = jnp.searchsorted(dst1d, tick + npt).astype(jnp.int32)
    bounds = jnp.concatenate(
        [starts[:, None], ends[:, None],
         jnp.zeros((16, 14), jnp.int32)], axis=1)

    BN = 1000
    nblk = n // BN

    sc_agg = _make_sc_agg(n, e_pad)

    init_call = pl.pallas_call(
        _init_body,
        grid=(nblk,),
        in_specs=[
            pl.BlockSpec((BN, d_in), lambda i: (i, 0)),
            pl.BlockSpec((d_in, h_dim), lambda i: (0, 0)),
            pl.BlockSpec((1, h_dim), lambda i: (0, 0)),
        ],
        out_specs=pl.BlockSpec((4, BN, 128), lambda i: (0, i, 0)),
        out_shape=jax.ShapeDtypeStruct((4, n, 128), jnp.float32),
    )

    stats_call = pl.pallas_call(
        _stats_body,
        grid=(nblk,),
        in_specs=[
            pl.BlockSpec((1, 1), lambda i: (0, 0)),
            pl.BlockSpec((4, BN, 128), lambda i: (0, i, 0)),
            pl.BlockSpec((4, BN, 128), lambda i: (0, i, 0)),
        ],
        out_specs=pl.BlockSpec((4, 8, 128), lambda i: (0, 0, 0)),
        out_shape=jax.ShapeDtypeStruct((4, 8, 128), jnp.float32),
    )

    def norm_call(last):
        return pl.pallas_call(
            functools.partial(_norm_body, float(n), last),
            grid=(nblk,),
            in_specs=[
                pl.BlockSpec((1, 1), lambda i: (0, 0)),
                pl.BlockSpec((4, 8, 128), lambda i: (0, 0, 0)),
                pl.BlockSpec((4, 1, 128), lambda i: (0, 0, 0)),
                pl.BlockSpec((4, 1, 128), lambda i: (0, 0, 0)),
                pl.BlockSpec((4, BN, 128), lambda i: (0, i, 0)),
                pl.BlockSpec((4, BN, 128), lambda i: (0, i, 0)),
            ],
            out_specs=[
                pl.BlockSpec((4, BN, 128), lambda i: (0, i, 0)),
                pl.BlockSpec((4, 8, 128), lambda i: (0, 0, 0)),
            ],
            out_shape=[
                jax.ShapeDtypeStruct((4, n, 128), jnp.float32),
                jax.ShapeDtypeStruct((4, 8, 128), jnp.float32),
            ],
        )

    head_call = pl.pallas_call(
        _head_body,
        in_specs=[
            pl.BlockSpec((4, 8, 128), lambda: (0, 0, 0)),
            pl.BlockSpec((4, 128, h_dim), lambda: (0, 0, 0)),
            pl.BlockSpec((1, h_dim), lambda: (0, 0)),
            pl.BlockSpec((h_dim, h_dim), lambda: (0, 0)),
            pl.BlockSpec((1, h_dim), lambda: (0, 0)),
            pl.BlockSpec((h_dim, n_out), lambda: (0, 0)),
            pl.BlockSpec((1, n_out), lambda: (0, 0)),
        ],
        out_specs=pl.BlockSpec((1, n_out), lambda: (0, 0)),
        out_shape=jax.ShapeDtypeStruct((1, n_out), jnp.float32),
    )

    epsp1 = (1.0 + eps).reshape(n_layers, 1, 1)
    gc = bn_gamma.reshape(n_layers, 4, 1, 128)
    bc = bn_beta.reshape(n_layers, 4, 1, 128)

    hc = init_call(x, W_init, b_init.reshape(1, h_dim))
    gacc = None
    for l in range(n_layers):
        agg = sc_agg(hc, src1d, dst1d, bounds)
        stats = stats_call(epsp1[l], hc, agg)
        hc, gacc = norm_call(l == n_layers - 1)(
            epsp1[l], stats, gc[l], bc[l], hc, agg)

    out = head_call(
        gacc,
        W_fc[0].reshape(4, 128, h_dim),
        b_fc[0].reshape(1, h_dim),
        W_fc[1],
        b_fc[1].reshape(1, h_dim),
        W_pred,
        b_pred.reshape(1, n_out),
    )
    return out.reshape(n_out)


# speedup vs baseline: 1.8868x; 1.8868x over previous
import functools

import jax
import jax.numpy as jnp
from jax import lax
from jax.experimental import pallas as pl
from jax.experimental.pallas import tpu as pltpu
from jax.experimental.pallas import tpu_sc as plsc


_CH = 128


def _make_sc_agg(n_nodes: int, n_edges_padded: int):
    CH = _CH
    NTILES = 16
    NG = n_edges_padded // (NTILES * CH)
    assert NG * NTILES * CH == n_edges_padded
    assert NG % 2 == 1
    NPT = ((n_nodes // NTILES + 7) // 8) * 8
    NROWS = NPT * NTILES
    LAST = n_nodes - NPT * (NTILES - 1)
    assert LAST > 0 and LAST % 8 == 0

    mesh = plsc.VectorSubcoreMesh(core_axis_name="c", subcore_axis_name="s")

    @functools.partial(
        pl.kernel,
        mesh=mesh,
        out_type=jax.ShapeDtypeStruct((4, n_nodes, 128), jnp.float32),
        scratch_types=[
            pltpu.VMEM((NG, CH), jnp.int32),
            pltpu.VMEM((CH,), jnp.int32),
            pltpu.VMEM((CH,), jnp.int32),
            pltpu.VMEM((CH, 128), jnp.float32),
            pltpu.VMEM((CH, 128), jnp.float32),
            pltpu.VMEM((8, 128), jnp.float32),
            pltpu.VMEM_SHARED((NROWS, 128), jnp.float32),
            pltpu.SemaphoreType.DMA,
            pltpu.SemaphoreType.DMA,
            pltpu.SemaphoreType.DMA,
            pltpu.SemaphoreType.DMA,
        ],
    )
    def sc_agg(h_hbm, src_hbm, dst_hbm, out_hbm,
               dstv, sidx0, sidx1, rows0, rows1, zbuf, aggsh,
               gsem0, gsem1, isem0, isem1):
        cid = lax.axis_index("c")
        sid = lax.axis_index("s")
        row_base = pl.multiple_of(sid * NPT, 8)
        gbase = sid * NG

        pltpu.sync_copy(dst_hbm.at[sid], dstv)

        def zb(j, carry):
            for k in range(8):
                zbuf[j, pl.ds(k * 16, 16)] = jnp.zeros((16,), jnp.float32)
            return carry
        lax.fori_loop(0, 8, zb, 0)

        def copy_rows(nrows, src_fn, dst_fn):
            full, rem = divmod(nrows, 80)
            for t in range(full):
                pltpu.sync_copy(src_fn(t * 80, 80), dst_fn(t * 80, 80))
            if rem:
                pltpu.sync_copy(src_fn(full * 80, rem), dst_fn(full * 80, rem))

        def _i_desc(j, buf, sem):
            return pltpu.make_async_copy(
                src_hbm.at[pl.ds((gbase + j) * CH, CH)], buf, sem)

        def do_chunk(f):
            def zero_body(j, carry):
                pltpu.sync_copy(
                    zbuf,
                    aggsh.at[pl.ds(pl.multiple_of(row_base + j * 8, 8), 8)])
                return carry
            lax.fori_loop(0, NPT // 8, zero_body, 0)
            plsc.subcore_barrier()

            def start_i(j, buf, sem):
                _i_desc(j, buf, sem).start()

            def wait_i(j, buf, sem):
                _i_desc(j, buf, sem).wait()

            def start_g(idxbuf, rbuf, sem):
                pltpu.async_copy(h_hbm.at[f].at[idxbuf], rbuf, sem)

            def wait_g(idxbuf, rbuf, sem):
                pltpu.make_async_copy(h_hbm.at[f].at[idxbuf], rbuf,
                                      sem).wait()

            def scat(rbuf, j):
                pltpu.sync_copy(rbuf, aggsh.at[dstv.at[j]], add=True)

            start_i(0, sidx0, isem0)
            wait_i(0, sidx0, isem0)
            start_g(sidx0, rows0, gsem0)
            start_i(1, sidx1, isem1)

            def acc2(i, carry):
                j0 = 2 * i
                wait_g(sidx0, rows0, gsem0)
                wait_i(j0 + 1, sidx1, isem1)
                start_g(sidx1, rows1, gsem1)
                start_i(j0 + 2, sidx0, isem0)
                scat(rows0, j0)
                wait_g(sidx1, rows1, gsem1)
                wait_i(j0 + 2, sidx0, isem0)
                start_g(sidx0, rows0, gsem0)
                start_i(jnp.minimum(j0 + 3, NG - 1), sidx1, isem1)
                scat(rows1, j0 + 1)
                return carry
            lax.fori_loop(0, (NG - 1) // 2, acc2, 0)

            wait_g(sidx0, rows0, gsem0)
            scat(rows0, NG - 1)
            wait_i(NG - 1, sidx1, isem1)
            plsc.subcore_barrier()

            def wb(nrows):
                copy_rows(
                    nrows,
                    lambda o, k: aggsh.at[pl.ds(row_base + o, k)],
                    lambda o, k: out_hbm.at[f].at[pl.ds(row_base + o, k)],
                )

            @pl.when(sid < NTILES - 1)
            def _():
                wb(NPT)

            @pl.when(sid == NTILES - 1)
            def _():
                wb(LAST)

        @pl.when(cid == 0)
        def _():
            do_chunk(0)
            do_chunk(1)

        @pl.when(cid == 1)
        def _():
            do_chunk(2)
            do_chunk(3)

    return sc_agg


def _init_body(x_ref, w_ref, b_ref, out_ref):
    h = jnp.dot(x_ref[...], w_ref[...], preferred_element_type=jnp.float32)
    h = h + b_ref[...]
    for f in range(4):
        out_ref[f] = h[:, 128 * f:128 * (f + 1)]


def _stats_body(ep_ref, h_ref, a_ref, stat_ref):
    u = ep_ref[...] * h_ref[...] + a_ref[...]
    s = jnp.sum(u, axis=1)
    q = jnp.sum(u * u, axis=1)

    @pl.when(pl.program_id(0) == 0)
    def _():
        stat_ref[...] = jnp.zeros_like(stat_ref)

    stat_ref[:, 0, :] += s
    stat_ref[:, 1, :] += q


def _norm_body(n_nodes, last, ep_ref, stat_ref, g_ref, bt_ref, h_ref, a_ref,
               out_ref, gacc_ref):
    u = ep_ref[...] * h_ref[...] + a_ref[...]
    mean = stat_ref[:, 0:1, :] / n_nodes
    var = stat_ref[:, 1:2, :] / n_nodes - mean * mean
    scale = g_ref[...] * lax.rsqrt(var + 1e-5)
    r = jnp.maximum((u - mean) * scale + bt_ref[...], 0.0)
    out_ref[...] = r

    if last:
        @pl.when(pl.program_id(0) == 0)
        def _():
            gacc_ref[...] = jnp.zeros_like(gacc_ref)
        gacc_ref[:, 0, :] += jnp.sum(r, axis=1)


def _head_body(gp_ref, w0_ref, b0_ref, w1_ref, b1_ref, wp_ref, bp_ref,
               out_ref):
    g = jnp.sum(gp_ref[...], axis=1)
    acc = jnp.zeros((1, w0_ref.shape[2]), jnp.float32)
    for f in range(4):
        acc = acc + jnp.dot(g[f:f + 1, :], w0_ref[f],
                            preferred_element_type=jnp.float32)
    h1 = jnp.maximum(acc + b0_ref[...], 0.0)
    h2 = jnp.maximum(jnp.dot(h1, w1_ref[...],
                             preferred_element_type=jnp.float32) + b1_ref[...],
                     0.0)
    out_ref[...] = jnp.dot(h2, wp_ref[...],
                           preferred_element_type=jnp.float32) + bp_ref[...]


def kernel(x, edge_index, W_init, b_init, eps, bn_gamma, bn_beta,
           W_fc, b_fc, W_pred, b_pred):
    n, d_in = x.shape
    h_dim = W_init.shape[1]
    n_layers = eps.shape[0]
    n_edges = edge_index.shape[1]
    n_out = W_pred.shape[1]
    assert h_dim == 512

    CH = _CH
    ng = -(-n_edges // (16 * CH))
    ng += 1 - ng % 2
    e_pad = 16 * CH * ng
    pad = e_pad - n_edges
    src1d = jnp.concatenate([edge_index[0], jnp.zeros((pad,), jnp.int32)])
    dst3d = jnp.concatenate(
        [edge_index[1], jnp.full((pad,), n, jnp.int32)]).reshape(16, ng, CH)

    BN = 1000
    nblk = n // BN

    sc_agg = _make_sc_agg(n, e_pad)

    init_call = pl.pallas_call(
        _init_body,
        grid=(nblk,),
        in_specs=[
            pl.BlockSpec((BN, d_in), lambda i: (i, 0)),
            pl.BlockSpec((d_in, h_dim), lambda i: (0, 0)),
            pl.BlockSpec((1, h_dim), lambda i: (0, 0)),
        ],
        out_specs=pl.BlockSpec((4, BN, 128), lambda i: (0, i, 0)),
        out_shape=jax.ShapeDtypeStruct((4, n, 128), jnp.float32),
    )

    stats_call = pl.pallas_call(
        _stats_body,
        grid=(nblk,),
        in_specs=[
            pl.BlockSpec((1, 1), lambda i: (0, 0)),
            pl.BlockSpec((4, BN, 128), lambda i: (0, i, 0)),
            pl.BlockSpec((4, BN, 128), lambda i: (0, i, 0)),
        ],
        out_specs=pl.BlockSpec((4, 8, 128), lambda i: (0, 0, 0)),
        out_shape=jax.ShapeDtypeStruct((4, 8, 128), jnp.float32),
    )

    def norm_call(last):
        return pl.pallas_call(
            functools.partial(_norm_body, float(n), last),
            grid=(nblk,),
            in_specs=[
                pl.BlockSpec((1, 1), lambda i: (0, 0)),
                pl.BlockSpec((4, 8, 128), lambda i: (0, 0, 0)),
                pl.BlockSpec((4, 1, 128), lambda i: (0, 0, 0)),
                pl.BlockSpec((4, 1, 128), lambda i: (0, 0, 0)),
                pl.BlockSpec((4, BN, 128), lambda i: (0, i, 0)),
                pl.BlockSpec((4, BN, 128), lambda i: (0, i, 0)),
            ],
            out_specs=[
                pl.BlockSpec((4, BN, 128), lambda i: (0, i, 0)),
                pl.BlockSpec((4, 8, 128), lambda i: (0, 0, 0)),
            ],
            out_shape=[
                jax.ShapeDtypeStruct((4, n, 128), jnp.float32),
                jax.ShapeDtypeStruct((4, 8, 128), jnp.float32),
            ],
        )

    head_call = pl.pallas_call(
        _head_body,
        in_specs=[
            pl.BlockSpec((4, 8, 128), lambda: (0, 0, 0)),
            pl.BlockSpec((4, 128, h_dim), lambda: (0, 0, 0)),
            pl.BlockSpec((1, h_dim), lambda: (0, 0)),
            pl.BlockSpec((h_dim, h_dim), lambda: (0, 0)),
            pl.BlockSpec((1, h_dim), lambda: (0, 0)),
            pl.BlockSpec((h_dim, n_out), lambda: (0, 0)),
            pl.BlockSpec((1, n_out), lambda: (0, 0)),
        ],
        out_specs=pl.BlockSpec((1, n_out), lambda: (0, 0)),
        out_shape=jax.ShapeDtypeStruct((1, n_out), jnp.float32),
    )

    epsp1 = (1.0 + eps).reshape(n_layers, 1, 1)
    gc = bn_gamma.reshape(n_layers, 4, 1, 128)
    bc = bn_beta.reshape(n_layers, 4, 1, 128)

    hc = init_call(x, W_init, b_init.reshape(1, h_dim))
    gacc = None
    for l in range(n_layers):
        agg = sc_agg(hc, src1d, dst3d)
        stats = stats_call(epsp1[l], hc, agg)
        hc, gacc = norm_call(l == n_layers - 1)(
            epsp1[l], stats, gc[l], bc[l], hc, agg)

    out = head_call(
        gacc,
        W_fc[0].reshape(4, 128, h_dim),
        b_fc[0].reshape(1, h_dim),
        W_fc[1],
        b_fc[1].reshape(1, h_dim),
        W_pred,
        b_pred.reshape(1, n_out),
    )
    return out.reshape(n_out)


# speedup vs baseline: 1.9285x; 1.0221x over previous
import functools

import jax
import jax.numpy as jnp
from jax import lax
from jax.experimental import pallas as pl
from jax.experimental.pallas import tpu as pltpu
from jax.experimental.pallas import tpu_sc as plsc


_CH = 80


def _make_sc_agg(n_nodes: int, n_edges_padded: int):
    CH = _CH
    NTILES = 16
    CPT = n_edges_padded // (NTILES * CH)
    assert CPT * NTILES * CH == n_edges_padded
    NPT = ((n_nodes // NTILES + 7) // 8) * 8
    NROWS = NPT * NTILES
    LAST = n_nodes - NPT * (NTILES - 1)
    assert LAST > 0 and LAST % 8 == 0

    mesh = plsc.VectorSubcoreMesh(core_axis_name="c", subcore_axis_name="s")

    @functools.partial(
        pl.kernel,
        mesh=mesh,
        out_type=jax.ShapeDtypeStruct((4, n_nodes, 128), jnp.float32),
        scratch_types=[
            pltpu.VMEM((CPT, CH), jnp.int32),
            pltpu.VMEM((CPT, CH), jnp.int32),
            pltpu.VMEM((CH, 128), jnp.float32),
            pltpu.VMEM((8, 128), jnp.float32),
            pltpu.VMEM_SHARED((NROWS, 128), jnp.float32),
            pltpu.SemaphoreType.DMA,
        ],
    )
    def sc_agg(h_hbm, src_hbm, dst_hbm, out_hbm,
               srcv, dstv, rows, zbuf, aggsh, sem):
        cid = lax.axis_index("c")
        sid = lax.axis_index("s")
        row_base = pl.multiple_of(sid * NPT, 8)

        pltpu.sync_copy(src_hbm.at[sid], srcv)
        pltpu.sync_copy(dst_hbm.at[sid], dstv)

        def zb(j, carry):
            for k in range(8):
                zbuf[j, pl.ds(k * 16, 16)] = jnp.zeros((16,), jnp.float32)
            return carry
        lax.fori_loop(0, 8, zb, 0)

        def copy_rows(nrows, src_fn, dst_fn):
            full, rem = divmod(nrows, 80)
            for t in range(full):
                pltpu.sync_copy(src_fn(t * 80, 80), dst_fn(t * 80, 80))
            if rem:
                pltpu.sync_copy(src_fn(full * 80, rem), dst_fn(full * 80, rem))

        def do_chunk(f):
            def zero_body(j, carry):
                pltpu.sync_copy(
                    zbuf,
                    aggsh.at[pl.ds(pl.multiple_of(row_base + j * 8, 8), 8)])
                return carry
            lax.fori_loop(0, NPT // 8, zero_body, 0)
            plsc.subcore_barrier()

            def acc(j, carry):
                pltpu.async_copy(h_hbm.at[f].at[srcv.at[j]], rows,
                                 sem).wait()
                pltpu.sync_copy(rows, aggsh.at[dstv.at[j]], add=True)
                return carry
            lax.fori_loop(0, CPT, acc, 0)
            plsc.subcore_barrier()

            def wb(nrows):
                copy_rows(
                    nrows,
                    lambda o, k: aggsh.at[pl.ds(row_base + o, k)],
                    lambda o, k: out_hbm.at[f].at[pl.ds(row_base + o, k)],
                )

            @pl.when(sid < NTILES - 1)
            def _():
                wb(NPT)

            @pl.when(sid == NTILES - 1)
            def _():
                wb(LAST)

        @pl.when(cid == 0)
        def _():
            do_chunk(0)
            do_chunk(1)

        @pl.when(cid == 1)
        def _():
            do_chunk(2)
            do_chunk(3)

    return sc_agg


def _init_body(x_ref, w_ref, b_ref, out_ref):
    h = jnp.dot(x_ref[...], w_ref[...], preferred_element_type=jnp.float32)
    h = h + b_ref[...]
    for f in range(4):
        out_ref[f] = h[:, 128 * f:128 * (f + 1)]


def _stats_body(ep_ref, h_ref, a_ref, stat_ref):
    u = ep_ref[...] * h_ref[...] + a_ref[...]
    s = jnp.sum(u, axis=1)
    q = jnp.sum(u * u, axis=1)

    @pl.when(pl.program_id(0) == 0)
    def _():
        stat_ref[...] = jnp.zeros_like(stat_ref)

    stat_ref[:, 0, :] += s
    stat_ref[:, 1, :] += q


def _norm_body(n_nodes, last, ep_ref, stat_ref, g_ref, bt_ref, h_ref, a_ref,
               out_ref, gacc_ref):
    u = ep_ref[...] * h_ref[...] + a_ref[...]
    mean = stat_ref[:, 0:1, :] / n_nodes
    var = stat_ref[:, 1:2, :] / n_nodes - mean * mean
    scale = g_ref[...] * lax.rsqrt(var + 1e-5)
    r = jnp.maximum((u - mean) * scale + bt_ref[...], 0.0)
    out_ref[...] = r

    if last:
        @pl.when(pl.program_id(0) == 0)
        def _():
            gacc_ref[...] = jnp.zeros_like(gacc_ref)
        gacc_ref[:, 0, :] += jnp.sum(r, axis=1)


def _head_body(gp_ref, w0_ref, b0_ref, w1_ref, b1_ref, wp_ref, bp_ref,
               out_ref):
    g = jnp.sum(gp_ref[...], axis=1)
    acc = jnp.zeros((1, w0_ref.shape[2]), jnp.float32)
    for f in range(4):
        acc = acc + jnp.dot(g[f:f + 1, :], w0_ref[f],
                            preferred_element_type=jnp.float32)
    h1 = jnp.maximum(acc + b0_ref[...], 0.0)
    h2 = jnp.maximum(jnp.dot(h1, w1_ref[...],
                             preferred_element_type=jnp.float32) + b1_ref[...],
                     0.0)
    out_ref[...] = jnp.dot(h2, wp_ref[...],
                           preferred_element_type=jnp.float32) + bp_ref[...]


def kernel(x, edge_index, W_init, b_init, eps, bn_gamma, bn_beta,
           W_fc, b_fc, W_pred, b_pred):
    n, d_in = x.shape
    h_dim = W_init.shape[1]
    n_layers = eps.shape[0]
    n_edges = edge_index.shape[1]
    n_out = W_pred.shape[1]
    assert h_dim == 512

    CH = _CH
    e_pad = -(-n_edges // (16 * CH)) * 16 * CH
    pad = e_pad - n_edges
    src3d = jnp.concatenate(
        [edge_index[0], jnp.zeros((pad,), jnp.int32)]).reshape(16, -1, CH)
    dst3d = jnp.concatenate(
        [edge_index[1], jnp.full((pad,), n, jnp.int32)]).reshape(16, -1, CH)

    BN = 1000
    nblk = n // BN

    sc_agg = _make_sc_agg(n, e_pad)

    init_call = pl.pallas_call(
        _init_body,
        grid=(nblk,),
        in_specs=[
            pl.BlockSpec((BN, d_in), lambda i: (i, 0)),
            pl.BlockSpec((d_in, h_dim), lambda i: (0, 0)),
            pl.BlockSpec((1, h_dim), lambda i: (0, 0)),
        ],
        out_specs=pl.BlockSpec((4, BN, 128), lambda i: (0, i, 0)),
        out_shape=jax.ShapeDtypeStruct((4, n, 128), jnp.float32),
    )

    stats_call = pl.pallas_call(
        _stats_body,
        grid=(nblk,),
        in_specs=[
            pl.BlockSpec((1, 1), lambda i: (0, 0)),
            pl.BlockSpec((4, BN, 128), lambda i: (0, i, 0)),
            pl.BlockSpec((4, BN, 128), lambda i: (0, i, 0)),
        ],
        out_specs=pl.BlockSpec((4, 8, 128), lambda i: (0, 0, 0)),
        out_shape=jax.ShapeDtypeStruct((4, 8, 128), jnp.float32),
    )

    def norm_call(last):
        return pl.pallas_call(
            functools.partial(_norm_body, float(n), last),
            grid=(nblk,),
            in_specs=[
                pl.BlockSpec((1, 1), lambda i: (0, 0)),
                pl.BlockSpec((4, 8, 128), lambda i: (0, 0, 0)),
                pl.BlockSpec((4, 1, 128), lambda i: (0, 0, 0)),
                pl.BlockSpec((4, 1, 128), lambda i: (0, 0, 0)),
                pl.BlockSpec((4, BN, 128), lambda i: (0, i, 0)),
                pl.BlockSpec((4, BN, 128), lambda i: (0, i, 0)),
            ],
            out_specs=[
                pl.BlockSpec((4, BN, 128), lambda i: (0, i, 0)),
                pl.BlockSpec((4, 8, 128), lambda i: (0, 0, 0)),
            ],
            out_shape=[
                jax.ShapeDtypeStruct((4, n, 128), jnp.float32),
                jax.ShapeDtypeStruct((4, 8, 128), jnp.float32),
            ],
        )

    head_call = pl.pallas_call(
        _head_body,
        in_specs=[
            pl.BlockSpec((4, 8, 128), lambda: (0, 0, 0)),
            pl.BlockSpec((4, 128, h_dim), lambda: (0, 0, 0)),
            pl.BlockSpec((1, h_dim), lambda: (0, 0)),
            pl.BlockSpec((h_dim, h_dim), lambda: (0, 0)),
            pl.BlockSpec((1, h_dim), lambda: (0, 0)),
            pl.BlockSpec((h_dim, n_out), lambda: (0, 0)),
            pl.BlockSpec((1, n_out), lambda: (0, 0)),
        ],
        out_specs=pl.BlockSpec((1, n_out), lambda: (0, 0)),
        out_shape=jax.ShapeDtypeStruct((1, n_out), jnp.float32),
    )

    epsp1 = (1.0 + eps).reshape(n_layers, 1, 1)
    gc = bn_gamma.reshape(n_layers, 4, 1, 128)
    bc = bn_beta.reshape(n_layers, 4, 1, 128)

    hc = init_call(x, W_init, b_init.reshape(1, h_dim))
    gacc = None
    for l in range(n_layers):
        agg = sc_agg(hc, src3d, dst3d)
        stats = stats_call(epsp1[l], hc, agg)
        hc, gacc = norm_call(l == n_layers - 1)(
            epsp1[l], stats, gc[l], bc[l], hc, agg)

    out = head_call(
        gacc,
        W_fc[0].reshape(4, 128, h_dim),
        b_fc[0].reshape(1, h_dim),
        W_fc[1],
        b_fc[1].reshape(1, h_dim),
        W_pred,
        b_pred.reshape(1, n_out),
    )
    return out.reshape(n_out)
